# baseline (speedup 1.0000x reference)
import functools

import jax
import jax.numpy as jnp
from jax import lax
from jax.experimental import pallas as pl
from jax.experimental.pallas import tpu as pltpu
from jax.experimental.pallas import tpu_sc as plsc

L = 16384
D = 32
PD = 128
NC = 2
NS = 16
NW = NC * NS
BPW = L // NW
CH = 128
NSTEP = BPW // CH

_MESH = plsc.VectorSubcoreMesh(core_axis_name="c", subcore_axis_name="s")


def _body(dp_h, p_h, dl_h, l_h, t0_h, t1_h, t2_h, t3_h, out_h,
          i0, i1, i2, i3, r0, r1, r2, r3, obuf, si0, si1, si2, si3, sg):
    wid = lax.axis_index("s") * NC + lax.axis_index("c")
    base = wid * BPW
    idx_hs = (dp_h, p_h, dl_h, l_h)
    tbl_hs = (t0_h, t1_h, t2_h, t3_h)
    ivs = (i0, i1, i2, i3)
    rvs = (r0, r1, r2, r3)
    sis = (si0, si1, si2, si3)

    ic = [pltpu.async_copy(idx_hs[c].at[pl.ds(base, BPW)], ivs[c], sis[c])
          for c in range(4)]
    for c in range(4):
        ic[c].wait()

    for j in range(NSTEP):
        gc = [pltpu.async_copy(
                  tbl_hs[c].at[ivs[c].at[pl.ds(j * CH, CH)]], rvs[c], sg)
              for c in range(4)]
        for c in range(4):
            gc[c].wait()

        def asm(jj, _):
            for c in range(4):
                obuf[jj, pl.ds(c * D, 16)] = rvs[c][jj, pl.ds(0, 16)]
                obuf[jj, pl.ds(c * D + 16, 16)] = rvs[c][jj, pl.ds(16, 16)]
            return 0

        lax.fori_loop(0, CH, asm, 0, unroll=False)
        pltpu.sync_copy(obuf, out_h.at[pl.ds(base + j * CH, CH), :])


@functools.partial(
    pl.kernel,
    mesh=_MESH,
    out_type=jax.ShapeDtypeStruct((L, 4 * D), jnp.float32),
    scratch_types=[
        pltpu.VMEM((BPW,), jnp.int32),
        pltpu.VMEM((BPW,), jnp.int32),
        pltpu.VMEM((BPW,), jnp.int32),
        pltpu.VMEM((BPW,), jnp.int32),
        pltpu.VMEM((CH, PD), jnp.float32),
        pltpu.VMEM((CH, PD), jnp.float32),
        pltpu.VMEM((CH, PD), jnp.float32),
        pltpu.VMEM((CH, PD), jnp.float32),
        pltpu.VMEM((CH, 4 * D), jnp.float32),
        pltpu.SemaphoreType.DMA,
        pltpu.SemaphoreType.DMA,
        pltpu.SemaphoreType.DMA,
        pltpu.SemaphoreType.DMA,
        pltpu.SemaphoreType.DMA,
    ],
)
def _hgnn_gather(*args):
    _body(*args)


def kernel(dp, p, dl, l, Edp_emb, Eddp_emb, Edl_emb, Eddl_emb):
    dp = dp.astype(jnp.int32)
    p = p.astype(jnp.int32)
    dl = dl.astype(jnp.int32)
    l = l.astype(jnp.int32)
    pad = ((0, 0), (0, PD - D))
    t0 = jnp.pad(Edp_emb, pad)
    t1 = jnp.pad(Eddp_emb, pad)
    t2 = jnp.pad(Edl_emb, pad)
    t3 = jnp.pad(Eddl_emb, pad)
    out = _hgnn_gather(dp, p, dl, l, t0, t1, t2, t3)
    return out.reshape(1, L, 4 * D)


# speedup vs baseline: 1.1745x; 1.1745x over previous
import functools

import jax
import jax.numpy as jnp
from jax import lax
from jax.experimental import pallas as pl
from jax.experimental.pallas import tpu as pltpu
from jax.experimental.pallas import tpu_sc as plsc

L = 16384
D = 32
NC = 2
NS = 16
NW = NC * NS
BPW = L // NW

_MESH = plsc.VectorSubcoreMesh(core_axis_name="c", subcore_axis_name="s")
_NOTC = pltpu.CompilerParams(use_tc_tiling_on_sc=False)


def _band_body(bands, idx_hs, tbl_hs, out_h, ivs, rvs, sis, sgs, sws):
    wid = lax.axis_index("s") * NC + lax.axis_index("c")
    base = wid * BPW
    n = len(bands)
    ic = [pltpu.async_copy(idx_hs[k].at[pl.ds(base, BPW)], ivs[k], sis[k])
          for k in range(n)]
    gc = []
    for k in range(n):
        ic[k].wait()
        gc.append(pltpu.async_copy(tbl_hs[k].at[ivs[k]], rvs[k], sgs[k]))
    wc = []
    for k in range(n):
        gc[k].wait()
        wc.append(pltpu.async_copy(
            rvs[k], out_h.at[pl.ds(base, BPW), pl.ds(bands[k] * D, D)],
            sws[k]))
    for k in range(n):
        wc[k].wait()


def _make_band_kernel(bands):
    n = len(bands)
    scratch = (
        [pltpu.VMEM((BPW,), jnp.int32)] * n
        + [pltpu.VMEM((BPW, D), jnp.float32)] * n
        + [pltpu.SemaphoreType.DMA] * (3 * n)
    )

    @functools.partial(
        pl.kernel,
        mesh=_MESH,
        out_type=(),
        scratch_types=scratch,
        compiler_params=_NOTC,
        name=f"hgnn_bands_{'_'.join(map(str, bands))}",
    )
    def band_kernel(*args):
        idx_hs = args[:n]
        tbl_hs = args[n:2 * n]
        out_h = args[2 * n]
        rest = args[2 * n + 1:]
        ivs = rest[:n]
        rvs = rest[n:2 * n]
        sis = rest[2 * n:3 * n]
        sgs = rest[3 * n:4 * n]
        sws = rest[4 * n:5 * n]
        _band_body(bands, idx_hs, tbl_hs, out_h, ivs, rvs, sis, sgs, sws)

    return band_kernel


_k_small = _make_band_kernel((0, 2))
_k_big1 = _make_band_kernel((1,))
_k_big3 = _make_band_kernel((3,))


def kernel(dp, p, dl, l, Edp_emb, Eddp_emb, Edl_emb, Eddl_emb):
    dp = dp.astype(jnp.int32)
    p = p.astype(jnp.int32)
    dl = dl.astype(jnp.int32)
    l = l.astype(jnp.int32)
    out_ref = jax.new_ref(jnp.empty((L, 4 * D), jnp.float32))
    _k_small(dp, dl, Edp_emb, Edl_emb, out_ref)
    _k_big1(p, Eddp_emb, out_ref)
    _k_big3(l, Eddl_emb, out_ref)
    return out_ref[...].reshape(1, L, 4 * D)
